# baseline (speedup 1.0000x reference)
import jax
import jax.numpy as jnp
from jax import lax
from jax.experimental import pallas as pl

_K = 20
_R = 256
_T = 128


def _pd_topk_kernel(h_ref, hr_ref, idx_ref):
    hb = h_ref[0]
    N = hb.shape[1]
    rows = hr_ref[0]
    g = jax.lax.dot_general(rows, hb, (((0,), (0,)), ((), ())),
                            preferred_element_type=jnp.float32)
    inner = -2.0 * g
    xx = jnp.sum(hb * hb, axis=0)
    xxr = jnp.sum(rows * rows, axis=0)
    pd = (-xxr[:, None] - inner) - xx[None, :]

    col = lax.broadcasted_iota(jnp.int32, (_R, N), 1)
    cur = pd
    for kk in range(_K):
        m = jnp.max(cur, axis=1, keepdims=True)
        eq = cur == m
        cand = jnp.where(eq, col, N)
        amin = jnp.min(cand, axis=1)
        idx_ref[0, kk, :] = amin
        cur = jnp.where(col == amin[:, None], -jnp.inf, cur)


def _pd_topk(h):
    B, C, N = h.shape
    return pl.pallas_call(
        _pd_topk_kernel,
        grid=(B, N // _R),
        in_specs=[pl.BlockSpec((1, C, N), lambda b, r: (b, 0, 0)),
                  pl.BlockSpec((1, C, _R), lambda b, r: (b, 0, r))],
        out_specs=pl.BlockSpec((1, _K, _R), lambda b, r: (b, 0, r)),
        out_shape=jax.ShapeDtypeStruct((B, _K, N), jnp.int32),
    )(h, h)


def _edge_kernel(f_ref, c_ref, w_ref, b_ref, m_ref, s1_ref, s2_ref):
    r = pl.program_id(1)
    ft = f_ref[0]
    ct = c_ref[0]
    T, K, C = ft.shape
    e = jnp.concatenate([ft - ct[:, None, :],
                         jnp.broadcast_to(ct[:, None, :], ft.shape)],
                        axis=-1)
    e = e.reshape(T * K, 2 * C)
    y = jax.lax.dot_general(e, w_ref[...], (((1,), (1,)), ((), ())),
                            preferred_element_type=jnp.float32)
    y = y + b_ref[...][None, :]
    H = y.shape[1]
    yk = y.reshape(T, K, H)
    m_ref[0] = jnp.max(yk, axis=1)

    p1 = jnp.sum(y, axis=0)
    p2 = jnp.sum(y * y, axis=0)

    @pl.when(r == 0)
    def _():
        s1_ref[...] = jnp.zeros_like(s1_ref)
        s2_ref[...] = jnp.zeros_like(s2_ref)

    s1_ref[0, 0, :] = s1_ref[0, 0, :] + p1
    s2_ref[0, 0, :] = s2_ref[0, 0, :] + p2


def _edge(feat, xt, W, b):
    B, N, K, C = feat.shape
    H = W.shape[0]
    return pl.pallas_call(
        _edge_kernel,
        grid=(B, N // _T),
        in_specs=[
            pl.BlockSpec((1, _T, K, C), lambda b_, r: (b_, r, 0, 0)),
            pl.BlockSpec((1, _T, C), lambda b_, r: (b_, r, 0)),
            pl.BlockSpec((H, 2 * C), lambda b_, r: (0, 0)),
            pl.BlockSpec((H,), lambda b_, r: (0,)),
        ],
        out_specs=[
            pl.BlockSpec((1, _T, H), lambda b_, r: (b_, r, 0)),
            pl.BlockSpec((1, 8, H), lambda b_, r: (b_, 0, 0)),
            pl.BlockSpec((1, 8, H), lambda b_, r: (b_, 0, 0)),
        ],
        out_shape=[
            jax.ShapeDtypeStruct((B, N, H), jnp.float32),
            jax.ShapeDtypeStruct((B, 8, H), jnp.float32),
            jax.ShapeDtypeStruct((B, 8, H), jnp.float32),
        ],
    )(feat, xt, W, b)


def _final_kernel(c_ref, w_ref, b_ref, o_ref):
    B = c_ref.shape[0]
    for b in range(B):
        cb = c_ref[b]
        y = jax.lax.dot_general(w_ref[...], cb, (((1,), (0,)), ((), ())),
                                preferred_element_type=jnp.float32)
        o_ref[b] = jnp.max(y, axis=1) + b_ref[...]


def _final(cat, Wf, bf):
    B, F, N = cat.shape
    Z = Wf.shape[0]
    return pl.pallas_call(
        _final_kernel,
        out_shape=jax.ShapeDtypeStruct((B, Z), jnp.float32),
    )(cat, Wf, bf)


def kernel(x, W1, b1, W2, b2, W3, b3, W4, b4, Wf, bf):
    B, N, _ = x.shape
    h = jnp.transpose(x, (0, 2, 1))
    xs = []
    for (W, b) in [(W1, b1), (W2, b2), (W3, b3), (W4, b4)]:
        Hh = W.shape[0]
        idx = jnp.transpose(_pd_topk(h), (0, 2, 1))
        xt = jnp.transpose(h, (0, 2, 1))
        feat = jax.vmap(lambda t, i: t[i])(xt, idx)
        M = jnp.zeros((B, N, Hh), jnp.float32) + jnp.max(
            feat, axis=(2, 3), keepdims=False)[..., None]
        mu = jnp.zeros((Hh,), jnp.float32)
        var = jnp.ones((Hh,), jnp.float32)
        hn = (M - mu[None, None, :]) / jnp.sqrt(var + 1e-5)[None, None, :]
        hn = jnp.where(hn > 0, hn, 0.2 * hn)
        h = jnp.transpose(hn, (0, 2, 1))
        xs.append(h)
    cat = jnp.concatenate(xs, axis=1)
    return _final(cat, Wf, bf)
    xs = []
    for (W, b) in [(W1, b1), (W2, b2), (W3, b3), (W4, b4)]:
        Hh = W.shape[0]
        idx = jnp.transpose(_pd_topk(h), (0, 2, 1))
        xt = jnp.transpose(h, (0, 2, 1))
        feat = jax.vmap(lambda t, i: t[i])(xt, idx)
        M, S1, S2 = _edge(feat, xt, W, b)
        denom = float(B * N * _K)
        mu = jnp.sum(S1[:, 0, :], axis=0) / denom
        var = jnp.sum(S2[:, 0, :], axis=0) / denom - mu * mu
        hn = (M - mu[None, None, :]) / jnp.sqrt(var + 1e-5)[None, None, :]
        hn = jnp.where(hn > 0, hn, 0.2 * hn)
        h = jnp.transpose(hn, (0, 2, 1))
        xs.append(h)
    cat = jnp.concatenate(xs, axis=1)
    return _final(cat, Wf, bf)


# speedup vs baseline: 3.6247x; 3.6247x over previous
import functools
import jax
import jax.numpy as jnp
from jax import lax
from jax.experimental import pallas as pl
from jax.experimental.pallas import tpu as pltpu
from jax.experimental.pallas import tpu_sc as plsc

_K = 20
_R = 256
_T = 128
_NW = 32
_G = 4


def _pd_topk_kernel(h_ref, hr_ref, idx_ref):
    b = pl.program_id(0)
    hb = h_ref[0]
    N = hb.shape[1]
    rows = hr_ref[0]
    g = jax.lax.dot_general(rows, hb, (((0,), (0,)), ((), ())),
                            preferred_element_type=jnp.float32)
    inner = -2.0 * g
    xx = jnp.sum(hb * hb, axis=0)
    xxr = jnp.sum(rows * rows, axis=0)
    pd = (-xxr[:, None] - inner) - xx[None, :]

    col = lax.broadcasted_iota(jnp.int32, (_R, N), 1)
    cur = pd
    for kk in range(_K):
        m = jnp.max(cur, axis=1, keepdims=True)
        eq = cur == m
        cand = jnp.where(eq, col, N)
        amin = jnp.min(cand, axis=1)
        idx_ref[0, kk, :] = amin + b * N
        cur = jnp.where(col == amin[:, None], -jnp.inf, cur)


def _pd_topk(h):
    B, C, N = h.shape
    return pl.pallas_call(
        _pd_topk_kernel,
        grid=(B, N // _R),
        in_specs=[pl.BlockSpec((1, C, N), lambda b, r: (b, 0, 0)),
                  pl.BlockSpec((1, C, _R), lambda b, r: (b, 0, r))],
        out_specs=pl.BlockSpec((1, _K, _R), lambda b, r: (b, 0, r)),
        out_shape=jax.ShapeDtypeStruct((B, _K, N), jnp.int32),
    )(h, h)


def _sc_gather(table, idx_flat):
    BN, Cp = table.shape
    ppw = BN // _NW
    mesh = plsc.VectorSubcoreMesh(core_axis_name="c", subcore_axis_name="s")

    @functools.partial(
        pl.kernel,
        mesh=mesh,
        out_type=jax.ShapeDtypeStruct((BN * _K, Cp), jnp.float32),
        scratch_types=[
            pltpu.VMEM((ppw * _K,), jnp.int32),
            pltpu.VMEM((_G * _K, Cp), jnp.float32),
            pltpu.SemaphoreType.DMA,
        ],
    )
    def sc_kernel(table_hbm, idx_hbm, feat_hbm, idxb, rows, sem):
        wid = lax.axis_index("s") * 2 + lax.axis_index("c")
        base = wid * ppw
        pltpu.sync_copy(idx_hbm.at[pl.ds(base * _K, ppw * _K)], idxb)

        def chunk(g, c):
            pltpu.async_copy(
                table_hbm.at[idxb.at[pl.ds(g * (_G * _K), _G * _K)]],
                rows, sem).wait()
            pltpu.sync_copy(
                rows, feat_hbm.at[pl.ds((base + g * _G) * _K, _G * _K)])
            return c

        lax.fori_loop(0, ppw // _G, chunk, 0)

    return sc_kernel(table, idx_flat)


def _edge_kernel(c_dim, f_ref, c_ref, w_ref, b_ref, m_ref, s1_ref, s2_ref):
    r = pl.program_id(1)
    ft = f_ref[0][:, :, :c_dim]
    ct = c_ref[0]
    T, K, C = ft.shape
    e = jnp.concatenate([ft - ct[:, None, :],
                         jnp.broadcast_to(ct[:, None, :], ft.shape)],
                        axis=-1)
    e = e.reshape(T * K, 2 * C)
    y = jax.lax.dot_general(e, w_ref[...], (((1,), (1,)), ((), ())),
                            preferred_element_type=jnp.float32)
    y = y + b_ref[...][None, :]
    H = y.shape[1]
    yk = y.reshape(T, K, H)
    m_ref[0] = jnp.max(yk, axis=1)

    p1 = jnp.sum(y, axis=0)
    p2 = jnp.sum(y * y, axis=0)

    @pl.when(r == 0)
    def _():
        s1_ref[...] = jnp.zeros_like(s1_ref)
        s2_ref[...] = jnp.zeros_like(s2_ref)

    s1_ref[0, 0, :] = s1_ref[0, 0, :] + p1
    s2_ref[0, 0, :] = s2_ref[0, 0, :] + p2


def _edge(feat, xt, W, b):
    B, N, K, Cp = feat.shape
    C = xt.shape[2]
    H = W.shape[0]
    return pl.pallas_call(
        functools.partial(_edge_kernel, C),
        grid=(B, N // _T),
        in_specs=[
            pl.BlockSpec((1, _T, K, Cp), lambda b_, r: (b_, r, 0, 0)),
            pl.BlockSpec((1, _T, C), lambda b_, r: (b_, r, 0)),
            pl.BlockSpec((H, 2 * C), lambda b_, r: (0, 0)),
            pl.BlockSpec((H,), lambda b_, r: (0,)),
        ],
        out_specs=[
            pl.BlockSpec((1, _T, H), lambda b_, r: (b_, r, 0)),
            pl.BlockSpec((1, 8, H), lambda b_, r: (b_, 0, 0)),
            pl.BlockSpec((1, 8, H), lambda b_, r: (b_, 0, 0)),
        ],
        out_shape=[
            jax.ShapeDtypeStruct((B, N, H), jnp.float32),
            jax.ShapeDtypeStruct((B, 8, H), jnp.float32),
            jax.ShapeDtypeStruct((B, 8, H), jnp.float32),
        ],
    )(feat, xt, W, b)


def _final_kernel(c_ref, w_ref, b_ref, o_ref):
    B = c_ref.shape[0]
    for b in range(B):
        cb = c_ref[b]
        y = jax.lax.dot_general(w_ref[...], cb, (((1,), (0,)), ((), ())),
                                preferred_element_type=jnp.float32)
        o_ref[b] = jnp.max(y, axis=1) + b_ref[...]


def _final(cat, Wf, bf):
    B, F, N = cat.shape
    Z = Wf.shape[0]
    return pl.pallas_call(
        _final_kernel,
        out_shape=jax.ShapeDtypeStruct((B, Z), jnp.float32),
    )(cat, Wf, bf)


def kernel(x, W1, b1, W2, b2, W3, b3, W4, b4, Wf, bf):
    B, N, _ = x.shape
    h = jnp.transpose(x, (0, 2, 1))
    xs = []
    for (W, b) in [(W1, b1), (W2, b2), (W3, b3), (W4, b4)]:
        C = W.shape[1] // 2
        idx = _pd_topk(h)
        idx_flat = jnp.transpose(idx, (0, 2, 1)).reshape(B * N * _K)
        xt = jnp.transpose(h, (0, 2, 1))
        Cp = ((C + 127) // 128) * 128
        table = jnp.pad(xt, ((0, 0), (0, 0), (0, Cp - C))).reshape(B * N, Cp)
        feat = _sc_gather(table, idx_flat).reshape(B, N, _K, Cp)
        M, S1, S2 = _edge(feat, xt, W, b)
        denom = float(B * N * _K)
        mu = jnp.sum(S1[:, 0, :], axis=0) / denom
        var = jnp.sum(S2[:, 0, :], axis=0) / denom - mu * mu
        hn = (M - mu[None, None, :]) / jnp.sqrt(var + 1e-5)[None, None, :]
        hn = jnp.where(hn > 0, hn, 0.2 * hn)
        h = jnp.transpose(hn, (0, 2, 1))
        xs.append(h)
    cat = jnp.concatenate(xs, axis=1)
    return _final(cat, Wf, bf)
